# baseline (speedup 1.0000x reference)
import jax
import jax.numpy as jnp
from jax import lax
from jax.experimental import pallas as pl
from jax.experimental.pallas import tpu as pltpu

_N_CODES = 1024
_CODE_DIM = 256
_BETA = 0.25
_NZB = 3
_NQB = 2


def _vq_body(z_hbm, cb_hbm, zq_hbm, idx_hbm, loss_ref,
             zbuf, qbuf, cbv, idxv, zsem, qsem, csem, isem):
    B, T, D = z_hbm.shape

    pltpu.make_async_copy(cb_hbm, cbv, csem).start()
    for j in range(min(_NZB, B)):
        pltpu.make_async_copy(z_hbm.at[j], zbuf.at[j], zsem.at[j]).start()
    pltpu.make_async_copy(cb_hbm, cbv, csem).wait()

    cb = cbv[...]
    csq = jnp.sum(cb * cb, axis=1)
    cols = lax.broadcasted_iota(jnp.int32, (T, _N_CODES), 1)

    total = jnp.zeros((), jnp.float32)
    for i in range(B):
        pltpu.make_async_copy(z_hbm.at[i], zbuf.at[i % _NZB],
                              zsem.at[i % _NZB]).wait()
        zb = zbuf[i % _NZB]

        s = lax.dot_general(zb, cb, (((1,), (1,)), ((), ())))
        zsq = jnp.sum(zb * zb, axis=1)
        dist = (zsq[:, None] - 2.0 * s) + csq[None, :]
        idx = jnp.argmin(dist, axis=1)

        oh = (cols == idx[:, None]).astype(jnp.float32)
        zq = lax.dot_general(oh, cb, (((1,), (0,)), ((), ())))

        if i >= _NQB:
            pltpu.make_async_copy(qbuf.at[i % _NQB], zq_hbm.at[i - _NQB],
                                  qsem.at[i % _NQB]).wait()
        qbuf[i % _NQB] = zq
        pltpu.make_async_copy(qbuf.at[i % _NQB], zq_hbm.at[i],
                              qsem.at[i % _NQB]).start()
        if i + _NZB < B:
            pltpu.make_async_copy(z_hbm.at[i + _NZB], zbuf.at[(i + _NZB) % _NZB],
                                  zsem.at[(i + _NZB) % _NZB]).start()

        idxv[i] = idx
        d = zb - zq
        total = total + jnp.sum(d * d)

    pltpu.make_async_copy(idxv, idx_hbm, isem).start()
    loss_ref[...] = total[None, None]
    for j in range(min(_NQB, B)):
        i = B - 1 - j
        pltpu.make_async_copy(qbuf.at[i % _NQB], zq_hbm.at[i],
                              qsem.at[i % _NQB]).wait()
    pltpu.make_async_copy(idxv, idx_hbm, isem).wait()


def kernel(z, codebook):
    B, D, H, W = z.shape
    hw = H * W
    z3 = z.transpose(0, 2, 3, 1).reshape(B, hw, D)

    zq, idx, loss = pl.pallas_call(
        _vq_body,
        in_specs=[
            pl.BlockSpec(memory_space=pl.ANY),
            pl.BlockSpec(memory_space=pl.ANY),
        ],
        out_specs=[
            pl.BlockSpec(memory_space=pl.ANY),
            pl.BlockSpec(memory_space=pl.ANY),
            pl.BlockSpec((1, 1), lambda: (0, 0)),
        ],
        out_shape=[
            jax.ShapeDtypeStruct((B, hw, D), jnp.float32),
            jax.ShapeDtypeStruct((B, hw), jnp.int32),
            jax.ShapeDtypeStruct((1, 1), jnp.float32),
        ],
        scratch_shapes=[
            pltpu.VMEM((_NZB, hw, D), jnp.float32),
            pltpu.VMEM((_NQB, hw, D), jnp.float32),
            pltpu.VMEM((_N_CODES, D), jnp.float32),
            pltpu.VMEM((B, hw), jnp.int32),
            pltpu.SemaphoreType.DMA((_NZB,)),
            pltpu.SemaphoreType.DMA((_NQB,)),
            pltpu.SemaphoreType.DMA,
            pltpu.SemaphoreType.DMA,
        ],
    )(z3, codebook)

    z_q_st = zq.reshape(B, H, W, D).transpose(0, 3, 1, 2)
    commitment_loss = loss[0, 0] * (_BETA / (B * hw * D))
    indices = idx.reshape(B, H, W)
    return z_q_st, commitment_loss, indices


# speedup vs baseline: 1.0975x; 1.0975x over previous
import jax
import jax.numpy as jnp
from jax import lax
from jax.experimental import pallas as pl
from jax.experimental.pallas import tpu as pltpu

_N_CODES = 1024
_CODE_DIM = 256
_BETA = 0.25


def _vq_body(z_ref, cb_ref, zq_ref, idx_ref, loss_ref):
    zb = z_ref[0]
    cb = cb_ref[...]
    T = zb.shape[0]

    s = lax.dot_general(zb, cb, (((1,), (1,)), ((), ())))
    zsq = jnp.sum(zb * zb, axis=1)
    csq = jnp.sum(cb * cb, axis=1)
    dist = (zsq[:, None] - 2.0 * s) + csq[None, :]
    idx = jnp.argmin(dist, axis=1)

    oh = (lax.broadcasted_iota(jnp.int32, (T, _N_CODES), 1)
          == idx[:, None]).astype(jnp.float32)
    zq = lax.dot_general(oh, cb, (((1,), (0,)), ((), ())))

    zq_ref[0] = zq
    idx_ref[0, 0] = idx

    d = zb - zq
    part = jnp.sum(d * d)
    first = pl.program_id(0) == 0

    @pl.when(first)
    def _():
        loss_ref[...] = part[None, None]

    @pl.when(jnp.logical_not(first))
    def _():
        loss_ref[...] += part[None, None]


def kernel(z, codebook):
    B, D, H, W = z.shape
    hw = H * W
    z3 = z.transpose(0, 2, 3, 1).reshape(B, hw, D)

    zq, idx, loss = pl.pallas_call(
        _vq_body,
        grid=(B,),
        in_specs=[
            pl.BlockSpec((1, hw, D), lambda i: (i, 0, 0)),
            pl.BlockSpec((_N_CODES, D), lambda i: (0, 0)),
        ],
        out_specs=[
            pl.BlockSpec((1, hw, D), lambda i: (i, 0, 0)),
            pl.BlockSpec((1, 1, hw), lambda i: (i, 0, 0)),
            pl.BlockSpec((1, 1), lambda i: (0, 0)),
        ],
        out_shape=[
            jax.ShapeDtypeStruct((B, hw, D), jnp.float32),
            jax.ShapeDtypeStruct((B, 1, hw), jnp.int32),
            jax.ShapeDtypeStruct((1, 1), jnp.float32),
        ],
        compiler_params=pltpu.CompilerParams(
            dimension_semantics=("arbitrary",)),
    )(z3, codebook)

    z_q_st = zq.reshape(B, H, W, D).transpose(0, 3, 1, 2)
    commitment_loss = loss[0, 0] * (_BETA / (B * hw * D))
    indices = idx.reshape(B, H, W)
    return z_q_st, commitment_loss, indices
